# baseline (speedup 1.0000x reference)
import functools

import jax
import jax.numpy as jnp
from jax.experimental import pallas as pl
from jax.experimental.pallas import tpu as pltpu

_BH = 16
_RC = 4
_KS = (3, 5, 7)


def _moe_block(xa_ref, rw1_ref, rb1_ref, rw2_ref, rb2_ref,
               dw0_ref, db0_ref, pw0_ref, pb0_ref,
               dw1_ref, db1_ref, pw1_ref, pb1_ref,
               dw2_ref, db2_ref, pw2_ref, pb2_ref,
               scale_ref, beta_ref, out_ref, sh_ref, *, w):
    c = out_ref.shape[1]
    bh = out_ref.shape[2]

    for dx in range(7):
        sh_ref[dx, bh:bh + 6] = sh_ref[dx, 0:6]

    xcur = xa_ref[0].astype(jnp.float32)
    for dx in range(7):
        sh_ref[dx, 0:bh] = xcur[:, dx:dx + w, :]

    dwt = (jnp.transpose(dw0_ref[...]), jnp.transpose(dw1_ref[...]),
           jnp.transpose(dw2_ref[...]))
    db_refs = (db0_ref, db1_ref, db2_ref)
    pw_refs = (pw0_ref, pw1_ref, pw2_ref)
    pb_refs = (pb0_ref, pb1_ref, pb2_ref)
    scl = scale_ref[...] * (1.0 / (1.0 + 1e-5)) ** 0.5
    dn = (((1,), (1,)), ((), ()))
    rc = _RC
    nc = rc * w
    for r in range(0, bh, rc):
        xc2 = sh_ref[3, 3 + r:3 + r + rc].reshape(nc, c)
        r1 = jnp.maximum(
            jax.lax.dot_general(xc2, rw1_ref[...], dn,
                                preferred_element_type=jnp.float32)
            + rb1_ref[...], 0.0)
        logits = (jax.lax.dot_general(r1, rw2_ref[...], dn,
                                      preferred_element_type=jnp.float32)
                  + rb2_ref[...])
        ex = jnp.exp(logits - jnp.max(logits, axis=1, keepdims=True))
        gate = ex / jnp.sum(ex, axis=1, keepdims=True)

        acc = None
        for e, k in enumerate(_KS):
            p = (k - 1) // 2
            d = None
            for adx in range(3 - p, 4 + p):
                u = sh_ref[adx, 3 - p + r:3 + p + r + rc]
                for dy in range(k):
                    t = u[dy:dy + rc] * dwt[e][dy * k + (adx - 3 + p)]
                    d = t if d is None else d + t
            de = (d + db_refs[e][...]).reshape(nc, c)
            pe = (jax.lax.dot_general(de, pw_refs[e][...], dn,
                                      preferred_element_type=jnp.float32)
                  + pb_refs[e][...])
            wp = pe * gate[:, e:e + 1]
            acc = wp if acc is None else acc + wp
        o = acc * scl + beta_ref[...]
        o = o * jax.nn.sigmoid(o)
        out_ref[0, :, r:r + rc, :] = jnp.transpose(
            o.reshape(rc, w, c), (2, 0, 1))


def kernel(x, rw1, rb1, rw2, rb2,
           dw_w0, dw_b0, pw_w0, pw_b0,
           dw_w1, dw_b1, pw_w1, pw_b1,
           dw_w2, dw_b2, pw_w2, pw_b2,
           bn_gamma, bn_beta):
    B, C, H, W = x.shape
    bh = _BH
    nj = H // bh
    f32 = jnp.float32
    red = rw1.shape[0]
    E = rw2.shape[0]

    xt = jnp.transpose(x.astype(jnp.bfloat16), (0, 2, 3, 1))
    xp = jnp.pad(xt, ((0, 0), (3, bh - 3), (3, 5), (0, 0)))

    rw1s = rw1.reshape(red, C)
    rb1s = rb1.reshape(1, red)
    rw2s = rw2.reshape(E, red)
    rb2s = rb2.reshape(1, E)
    dws = [dw_w0.reshape(C, 9), dw_w1.reshape(C, 25), dw_w2.reshape(C, 49)]
    dbs = [dw_b0.reshape(1, C), dw_b1.reshape(1, C), dw_b2.reshape(1, C)]
    pws = [pw_w0.reshape(C, C), pw_w1.reshape(C, C), pw_w2.reshape(C, C)]
    pbs = [pw_b0.reshape(1, C), pw_b1.reshape(1, C), pw_b2.reshape(1, C)]
    scale = bn_gamma.reshape(1, C)
    beta = bn_beta.reshape(1, C)

    Wp = W + 8
    rep = lambda b, j: (0, 0)
    specs = [
        pl.BlockSpec((1, bh, Wp, C), lambda b, j: (b, nj - j, 0, 0)),
        pl.BlockSpec((red, C), rep), pl.BlockSpec((1, red), rep),
        pl.BlockSpec((E, red), rep), pl.BlockSpec((1, E), rep),
    ]
    ops = [xp, rw1s, rb1s, rw2s, rb2s]
    for i, kk in enumerate((9, 25, 49)):
        specs += [pl.BlockSpec((C, kk), rep), pl.BlockSpec((1, C), rep),
                  pl.BlockSpec((C, C), rep), pl.BlockSpec((1, C), rep)]
        ops += [dws[i], dbs[i], pws[i], pbs[i]]
    specs += [pl.BlockSpec((1, C), rep), pl.BlockSpec((1, C), rep)]
    ops += [scale, beta]

    out = pl.pallas_call(
        functools.partial(_moe_block, w=W),
        grid=(B, nj + 1),
        in_specs=specs,
        out_specs=pl.BlockSpec(
            (1, C, bh, W),
            lambda b, j: (b, 0, jnp.minimum(nj - j, nj - 1), 0)),
        out_shape=jax.ShapeDtypeStruct((B, C, H, W), f32),
        scratch_shapes=[pltpu.VMEM((7, bh + 6, W, C), f32)],
    )(*ops)
    return out


# speedup vs baseline: 1.1176x; 1.1176x over previous
import functools

import jax
import jax.numpy as jnp
from jax.experimental import pallas as pl
from jax.experimental.pallas import tpu as pltpu

_BH = 16
_RC = 4
_KS = (3, 5, 7)


def _moe_block(xa_ref, rw1_ref, rb1_ref, rw2_ref, rb2_ref,
               dw0_ref, db0_ref, pw0_ref, pb0_ref,
               dw1_ref, db1_ref, pw1_ref, pb1_ref,
               dw2_ref, db2_ref, pw2_ref, pb2_ref,
               scale_ref, beta_ref, out_ref, sh_ref, *, w):
    c = out_ref.shape[1]
    bh = out_ref.shape[2]

    for dx in range(7):
        sh_ref[dx, bh:bh + 6] = sh_ref[dx, 0:6]

    xcur = xa_ref[0].astype(jnp.float32)
    for dx in range(7):
        sh_ref[dx, 0:bh] = xcur[:, dx:dx + w, :]

    dw_refs = (dw0_ref, dw1_ref, dw2_ref)
    db_refs = (db0_ref, db1_ref, db2_ref)
    pw_refs = (pw0_ref, pw1_ref, pw2_ref)
    pb_refs = (pb0_ref, pb1_ref, pb2_ref)
    rc = _RC
    nc = rc * w
    for r in range(0, bh, rc):
        xc2 = sh_ref[3, 3 + r:3 + r + rc].reshape(nc, c)
        r1 = jnp.maximum(
            jnp.dot(xc2, rw1_ref[...], preferred_element_type=jnp.float32)
            + rb1_ref[...], 0.0)
        logits = (jnp.dot(r1, rw2_ref[...], preferred_element_type=jnp.float32)
                  + rb2_ref[...])
        lane = jax.lax.broadcasted_iota(jnp.int32, logits.shape, 1)
        logits = jnp.where(lane < 3, logits, -1e30)
        ex = jnp.exp(logits - jnp.max(logits, axis=1, keepdims=True))
        gate = ex / jnp.sum(ex, axis=1, keepdims=True)

        acc = None
        for e, k in enumerate(_KS):
            p = (k - 1) // 2
            d = None
            pm = None
            for adx in range(3 - p, 4 + p):
                u = sh_ref[adx, 3 - p + r:3 + p + r + rc]
                for dy in range(k):
                    wv = dw_refs[e][dy * k + (adx - 3 + p)]
                    if e == 2 and (dy in (0, 6) or adx in (0, 6)):
                        s2 = u[dy:dy + rc].reshape(nc, c)
                        mt = pw_refs[e][...] * wv[:, None]
                        t2 = jnp.dot(s2, mt,
                                     preferred_element_type=jnp.float32)
                        pm = t2 if pm is None else pm + t2
                    else:
                        t = u[dy:dy + rc] * wv
                        d = t if d is None else d + t
            de = (d + db_refs[e][...]).reshape(nc, c)
            pe = (jnp.dot(de, pw_refs[e][...],
                          preferred_element_type=jnp.float32)
                  + pb_refs[e][...])
            if pm is not None:
                pe = pe + pm
            wp = pe * gate[:, e:e + 1]
            acc = wp if acc is None else acc + wp
        o = acc * scale_ref[...] + beta_ref[...]
        o = o * jax.nn.sigmoid(o)
        out_ref[0, :, r:r + rc, :] = jnp.transpose(
            o.reshape(rc, w, c), (2, 0, 1))


def kernel(x, rw1, rb1, rw2, rb2,
           dw_w0, dw_b0, pw_w0, pw_b0,
           dw_w1, dw_b1, pw_w1, pw_b1,
           dw_w2, dw_b2, pw_w2, pw_b2,
           bn_gamma, bn_beta):
    B, C, H, W = x.shape
    bh = _BH
    nj = H // bh
    f32 = jnp.float32
    red = rw1.shape[0]
    E = rw2.shape[0]

    xt = jnp.transpose(x.astype(jnp.bfloat16), (0, 2, 3, 1))
    xp = jnp.pad(xt, ((0, 0), (3, bh - 3), (3, 5), (0, 0)))

    rw1t = jnp.zeros((C, 128), f32).at[:, :red].set(rw1[:, :, 0, 0].T)
    rb1t = jnp.zeros((1, 128), f32).at[0, :red].set(rb1)
    rw2t = jnp.zeros((128, 8), f32).at[:red, :E].set(rw2[:, :, 0, 0].T)
    rb2t = jnp.zeros((1, 8), f32).at[0, :E].set(rb2)
    dwts, dbs, pwts, pbs = [], [], [], []
    for dw_w, dw_b, pw_w, pw_b, k in ((dw_w0, dw_b0, pw_w0, pw_b0, 3),
                                      (dw_w1, dw_b1, pw_w1, pw_b1, 5),
                                      (dw_w2, dw_b2, pw_w2, pw_b2, 7)):
        t = jnp.transpose(dw_w[:, 0, :, :], (1, 2, 0)).reshape(k * k, C)
        dwts.append(jnp.zeros((56, C), f32).at[:k * k, :].set(t))
        dbs.append(dw_b.reshape(1, C))
        pwts.append(pw_w[:, :, 0, 0].T)
        pbs.append(pw_b.reshape(1, C))
    scale = (bn_gamma / jnp.sqrt(1.0 + 1e-5)).reshape(1, C)
    beta = bn_beta.reshape(1, C)

    Wp = W + 8
    rep = lambda b, j: (0, 0)
    specs = [
        pl.BlockSpec((1, bh, Wp, C), lambda b, j: (b, nj - j, 0, 0)),
        pl.BlockSpec((C, 128), rep), pl.BlockSpec((1, 128), rep),
        pl.BlockSpec((128, 8), rep), pl.BlockSpec((1, 8), rep),
    ]
    ops = [xp, rw1t, rb1t, rw2t, rb2t]
    for i in range(3):
        specs += [pl.BlockSpec((56, C), rep), pl.BlockSpec((1, C), rep),
                  pl.BlockSpec((C, C), rep), pl.BlockSpec((1, C), rep)]
        ops += [dwts[i], dbs[i], pwts[i], pbs[i]]
    specs += [pl.BlockSpec((1, C), rep), pl.BlockSpec((1, C), rep)]
    ops += [scale, beta]

    out = pl.pallas_call(
        functools.partial(_moe_block, w=W),
        grid=(B, nj + 1),
        in_specs=specs,
        out_specs=pl.BlockSpec(
            (1, C, bh, W),
            lambda b, j: (b, 0, jnp.minimum(nj - j, nj - 1), 0)),
        out_shape=jax.ShapeDtypeStruct((B, C, H, W), f32),
        scratch_shapes=[pltpu.VMEM((7, bh + 6, W, C), f32)],
    )(*ops)
    return out
